# baseline (speedup 1.0000x reference)
import jax
import jax.numpy as jnp
from jax import lax
from jax.experimental import pallas as pl
from jax.experimental.pallas import tpu as pltpu
from jax.experimental.pallas import tpu_sc as plsc

NC = 2
NS = 16
LANES = 16
NW = NC * NS
DBLK = 8


def _body(embs_hbm, seq_hbm, pos_hbm, out_hbm, pos_v, seq_v, buf, in_sem, out_sem):
    seq_len, d_model, batch = embs_hbm.shape
    units = seq_len * (d_model // DBLK)
    per_w = units // NW
    bgroups = batch // LANES
    wid = lax.axis_index("s") * NC + lax.axis_index("c")

    pltpu.sync_copy(pos_hbm.at[:, pl.ds(0, pos_v.shape[1])], pos_v)
    pltpu.sync_copy(seq_hbm, seq_v)

    lane_iota = lax.iota(jnp.int32, LANES)

    def unit_idx(t):
        u = wid + t * NW
        return u // DBLK, (u % DBLK) * DBLK

    def in_copy(t, b):
        j, d0 = unit_idx(t)
        return pltpu.make_async_copy(
            embs_hbm.at[j, pl.ds(d0, DBLK)], buf.at[b], in_sem.at[b]
        )

    def out_copy(t, b):
        j, d0 = unit_idx(t)
        return pltpu.make_async_copy(
            buf.at[b], out_hbm.at[j, pl.ds(d0, DBLK)], out_sem.at[b]
        )

    in_copy(0, 0).start()

    def step(s, carry):
        for phase in range(2):
            b = phase
            t = s * 2 + phase
            j, d0 = unit_idx(t)

            def start_next():
                def retire_prev():
                    out_copy(t - 1, 1 - b).wait()

                if phase == 0:
                    pl.when(s >= 1)(retire_prev)
                else:
                    retire_prev()
                in_copy(t + 1, 1 - b).start()

            if phase == 1:
                pl.when(s < per_w // 2 - 1)(start_next)
            else:
                start_next()

            jp1 = j + 1
            lane = jp1 % LANES
            lbase = pl.multiple_of(jp1 - lane, LANES)
            lane_vec = jnp.broadcast_to(lane, (LANES,))
            dnums = lax.GatherDimensionNumbers(
                offset_dims=(), collapsed_slice_dims=(0,), start_index_map=(0,)
            )
            p_splat = []
            for dd in range(DBLK):
                row16 = pos_v[d0 + dd, pl.ds(lbase, LANES)]
                p_splat.append(
                    lax.gather(
                        row16,
                        lane_vec[:, None],
                        dnums,
                        slice_sizes=(1,),
                        mode=lax.GatherScatterMode.PROMISE_IN_BOUNDS,
                    )
                )

            in_copy(t, b).wait()

            @plsc.parallel_loop(0, bgroups, unroll=2)
            def bg_body(bg, _b=b, _j=j, _p=p_splat):
                sl = pl.ds(bg * LANES, LANES)
                m = seq_v[sl] > _j
                zero = jnp.zeros((LANES,), jnp.float32)
                for dd in range(DBLK):
                    plsc.addupdate(buf.at[_b, dd, sl], jnp.where(m, _p[dd], zero))

            out_copy(t, b).start()
        return carry

    lax.fori_loop(0, per_w // 2, step, 0)

    out_copy(per_w - 2, 0).wait()
    out_copy(per_w - 1, 1).wait()


@jax.jit
def kernel(embs, seq_lengths, pos_table):
    batch, seq_len, d_model = embs.shape
    embs_t = jnp.transpose(embs, (1, 2, 0))
    pos_t = jnp.transpose(pos_table, (1, 0))
    mesh = plsc.VectorSubcoreMesh(
        core_axis_name="c", subcore_axis_name="s", num_cores=NC, num_subcores=NS
    )
    pos_cols = seq_len + 1
    pos_cols += (-pos_cols) % 128
    run = pl.kernel(
        _body,
        out_type=jax.ShapeDtypeStruct((seq_len, d_model, batch), embs.dtype),
        mesh=mesh,
        compiler_params=pltpu.CompilerParams(use_tc_tiling_on_sc=True),
        scratch_types=[
            pltpu.VMEM((d_model, pos_cols), jnp.float32),
            pltpu.VMEM((batch,), jnp.int32),
            pltpu.VMEM((2, DBLK, batch), jnp.float32),
            pltpu.SemaphoreType.DMA((2,)),
            pltpu.SemaphoreType.DMA((2,)),
        ],
    )
    out_t = run(embs_t, seq_lengths.astype(jnp.int32), pos_t)
    return jnp.transpose(out_t, (2, 0, 1))


# speedup vs baseline: 1.0067x; 1.0067x over previous
import jax
import jax.numpy as jnp
from jax import lax
from jax.experimental import pallas as pl
from jax.experimental.pallas import tpu as pltpu
from jax.experimental.pallas import tpu_sc as plsc

NC = 2
NS = 16
LANES = 16
NW = NC * NS
DBLK = 8


def _body(embs_hbm, seq_hbm, pos_hbm, out_hbm, pos_v, seq_v, buf, in_sem, out_sem):
    seq_len, d_model, batch = embs_hbm.shape
    units = seq_len * (d_model // DBLK)
    per_w = units // NW
    bgroups = batch // LANES
    wid = lax.axis_index("s") * NC + lax.axis_index("c")

    pltpu.sync_copy(pos_hbm.at[:, pl.ds(0, pos_v.shape[1])], pos_v)
    pltpu.sync_copy(seq_hbm, seq_v)

    lane_iota = lax.iota(jnp.int32, LANES)

    def unit_idx(t):
        u = wid + t * NW
        return u // DBLK, (u % DBLK) * DBLK

    def in_copy(t, b):
        j, d0 = unit_idx(t)
        return pltpu.make_async_copy(
            embs_hbm.at[j, pl.ds(d0, DBLK)], buf.at[b], in_sem.at[b]
        )

    def out_copy(t, b):
        j, d0 = unit_idx(t)
        return pltpu.make_async_copy(
            buf.at[b], out_hbm.at[j, pl.ds(d0, DBLK)], out_sem.at[b]
        )

    in_copy(0, 0).start()

    def step(s, carry):
        for phase in range(2):
            b = phase
            t = s * 2 + phase
            j, d0 = unit_idx(t)

            def start_next():
                def retire_prev():
                    out_copy(t - 1, 1 - b).wait()

                if phase == 0:
                    pl.when(s >= 1)(retire_prev)
                else:
                    retire_prev()
                in_copy(t + 1, 1 - b).start()

            if phase == 1:
                pl.when(s < per_w // 2 - 1)(start_next)
            else:
                start_next()

            jp1 = j + 1
            lane = jp1 % LANES
            lbase = pl.multiple_of(jp1 - lane, LANES)
            lane_vec = jnp.broadcast_to(lane, (LANES,))
            dnums = lax.GatherDimensionNumbers(
                offset_dims=(), collapsed_slice_dims=(0,), start_index_map=(0,)
            )
            p_splat = []
            for dd in range(DBLK):
                row16 = pos_v[d0 + dd, pl.ds(lbase, LANES)]
                p_splat.append(
                    lax.gather(
                        row16,
                        lane_vec[:, None],
                        dnums,
                        slice_sizes=(1,),
                        mode=lax.GatherScatterMode.PROMISE_IN_BOUNDS,
                    )
                )

            in_copy(t, b).wait()

            @plsc.parallel_loop(0, bgroups, unroll=4)
            def bg_body(bg, _b=b, _j=j, _p=p_splat):
                sl = pl.ds(bg * LANES, LANES)
                m = seq_v[sl] > _j
                zero = jnp.zeros((LANES,), jnp.float32)
                for dd in range(DBLK):
                    plsc.addupdate(buf.at[_b, dd, sl], jnp.where(m, _p[dd], zero))

            out_copy(t, b).start()
        return carry

    lax.fori_loop(0, per_w // 2, step, 0)

    out_copy(per_w - 2, 0).wait()
    out_copy(per_w - 1, 1).wait()


@jax.jit
def kernel(embs, seq_lengths, pos_table):
    batch, seq_len, d_model = embs.shape
    embs_t = jnp.transpose(embs, (1, 2, 0))
    pos_t = jnp.transpose(pos_table, (1, 0))
    mesh = plsc.VectorSubcoreMesh(
        core_axis_name="c", subcore_axis_name="s", num_cores=NC, num_subcores=NS
    )
    pos_cols = seq_len + 1
    pos_cols += (-pos_cols) % 128
    run = pl.kernel(
        _body,
        out_type=jax.ShapeDtypeStruct((seq_len, d_model, batch), embs.dtype),
        mesh=mesh,
        compiler_params=pltpu.CompilerParams(use_tc_tiling_on_sc=True),
        scratch_types=[
            pltpu.VMEM((d_model, pos_cols), jnp.float32),
            pltpu.VMEM((batch,), jnp.int32),
            pltpu.VMEM((2, DBLK, batch), jnp.float32),
            pltpu.SemaphoreType.DMA((2,)),
            pltpu.SemaphoreType.DMA((2,)),
        ],
    )
    out_t = run(embs_t, seq_lengths.astype(jnp.int32), pos_t)
    return jnp.transpose(out_t, (2, 0, 1))
